# baseline (speedup 1.0000x reference)
import functools

import jax
import jax.numpy as jnp
import numpy as np
from jax.experimental import pallas as pl

B, S, D = 4, 4096, 2048
MASK_PCT = 0.15
S_BLK = 512

_k1, _k2 = jax.random.split(jax.random.key(1))
_scores_ne = jax.random.uniform(_k1, (B, S))
_scores_all = jax.random.uniform(_k2, (B, S))
PERM_NE = np.asarray(jnp.argsort(_scores_ne, axis=1), dtype=np.int32)
RANKS_ALL = np.asarray(
    jnp.argsort(jnp.argsort(_scores_all, axis=1), axis=1), dtype=np.int32
)
del _k1, _k2, _scores_ne, _scores_all


def _reduce_body(x_ref, ne_ref):
    x = x_ref[...]
    ne = jnp.any(x != 0.0, axis=-1)
    ne_ref[...] = ne[:, None, :].astype(jnp.float32)


def _nonempty_flags(data):
    grid = (B, S // S_BLK)
    return pl.pallas_call(
        _reduce_body,
        grid=grid,
        in_specs=[pl.BlockSpec((1, S_BLK, D), lambda b, t: (b, t, 0))],
        out_specs=pl.BlockSpec((1, 1, S_BLK), lambda b, t: (b, 0, t)),
        out_shape=jax.ShapeDtypeStruct((B, 1, S), jnp.float32),
    )(data).reshape(B, S)


def _apply_body(x_ref, m_ref, out_ref, mask_ref):
    x = x_ref[...]
    m = m_ref[...]
    mb = jnp.broadcast_to(m, x.shape)
    out_ref[...] = (1.0 - mb) * x
    mask_ref[...] = mb


def _apply_mask(data, mask2d):
    m3 = mask2d.reshape(B, S, 1)
    grid = (B, S // S_BLK)
    return pl.pallas_call(
        _apply_body,
        grid=grid,
        in_specs=[
            pl.BlockSpec((1, S_BLK, D), lambda b, t: (b, t, 0)),
            pl.BlockSpec((1, S_BLK, 1), lambda b, t: (b, t, 0)),
        ],
        out_specs=[
            pl.BlockSpec((1, S_BLK, D), lambda b, t: (b, t, 0)),
            pl.BlockSpec((1, S_BLK, D), lambda b, t: (b, t, 0)),
        ],
        out_shape=[
            jax.ShapeDtypeStruct((B, S, D), jnp.float32),
            jax.ShapeDtypeStruct((B, S, D), jnp.float32),
        ],
    )(data, m3)


def _mask2d_from_flags(ne):
    perm = jnp.asarray(PERM_NE)
    ranks_all = jnp.asarray(RANKS_ALL)
    count_ne = jnp.sum(ne, axis=1)
    k_ne = (count_ne * MASK_PCT).astype(jnp.int32)
    k_e = ((S - count_ne) * 0.1).astype(jnp.int32)
    g = jnp.take_along_axis(ne, perm, axis=1)
    csum = jnp.cumsum(g, axis=1) - g
    sel = (g > 0.0) & (csum < k_ne[:, None].astype(jnp.float32))
    mask_sorted = sel.astype(jnp.float32)
    inv = jnp.argsort(perm, axis=1)
    mask_ne = jnp.take_along_axis(mask_sorted, inv, axis=1)
    mask_e = (ranks_all < k_e[:, None]).astype(jnp.float32)
    return jnp.maximum(mask_ne, mask_e)


def kernel(data):
    ne = _nonempty_flags(data)
    mask2d = _mask2d_from_flags(ne)
    return _apply_mask(data, mask2d)


# speedup vs baseline: 1.4337x; 1.4337x over previous
import jax
import jax.numpy as jnp
import numpy as np
from jax.experimental import pallas as pl
from jax.experimental.pallas import tpu as pltpu

B, S, D = 4, 4096, 2048
MASK_PCT = 0.15
S_BLK = 512
T = S // S_BLK

_k1, _k2 = jax.random.split(jax.random.key(1))
_scores_ne = jax.random.uniform(_k1, (B, S))
_scores_all = jax.random.uniform(_k2, (B, S))


def _ranks_t(scores):
    r = np.asarray(
        jnp.argsort(jnp.argsort(scores, axis=1), axis=1), dtype=np.int32
    )
    return np.ascontiguousarray(r.reshape(B, T, S_BLK).transpose(0, 2, 1))


RANK_BASE_T = _ranks_t(_scores_ne)
RANKS_ALL_T = _ranks_t(_scores_all)
del _k1, _k2, _scores_ne, _scores_all


def _body(x_ref, rb_ref, ra_ref, out_ref, mask_ref, data_scr, ne_scr, m_scr):
    p = pl.program_id(1)
    t = pl.program_id(2)

    @pl.when(p == 0)
    def _reduce_phase():
        x = x_ref[0]
        data_scr[pl.ds(t * S_BLK, S_BLK), :] = x
        ne = jnp.any(x != 0.0, axis=-1).astype(jnp.float32)
        lane = jax.lax.broadcasted_iota(jnp.int32, (S_BLK, T), 1)
        ne_scr[...] = jnp.where(lane == t, ne[:, None], ne_scr[...])

    @pl.when((p == 1) & (t == 0))
    def _select_phase():
        ne = ne_scr[...]
        rank_base = rb_ref[0]
        ranks_all = ra_ref[0]
        count = jnp.sum(ne)
        k_ne = (count * MASK_PCT).astype(jnp.int32)
        k_e = ((S - count) * 0.1).astype(jnp.int32)

        k_ne_f = k_ne.astype(jnp.float32)

        def bis(_, lh):
            lo, hi = lh
            mid = (lo + hi) // 2
            n = jnp.sum(ne * (rank_base < mid).astype(jnp.float32))
            pred = n >= k_ne_f
            return (jnp.where(pred, lo, mid + 1), jnp.where(pred, mid, hi))

        lo, _ = jax.lax.fori_loop(
            0, 13, bis, (jnp.int32(0), jnp.int32(S)), unroll=True
        )
        mask_row = jnp.maximum(
            ne * (rank_base < lo).astype(jnp.float32),
            (ranks_all < k_e).astype(jnp.float32),
        )
        m_scr[...] = mask_row

    @pl.when(p == 1)
    def _apply_phase():
        xm = data_scr[pl.ds(t * S_BLK, S_BLK), :]
        lane = jax.lax.broadcasted_iota(jnp.int32, (S_BLK, T), 1)
        m = jnp.sum(m_scr[...] * (lane == t), axis=1, keepdims=True)
        mb = jnp.broadcast_to(m, (S_BLK, D))
        out_ref[...] = ((1.0 - mb) * xm)[None]
        mask_ref[...] = mb[None]


def kernel(data):
    out_map = lambda b, p, t: (b, t * p, 0)
    return pl.pallas_call(
        _body,
        grid=(B, 2, T),
        in_specs=[
            pl.BlockSpec((1, S_BLK, D), lambda b, p, t: (b, t + p * (T - 1 - t), 0)),
            pl.BlockSpec((1, S_BLK, T), lambda b, p, t: (b, 0, 0)),
            pl.BlockSpec((1, S_BLK, T), lambda b, p, t: (b, 0, 0)),
        ],
        out_specs=[
            pl.BlockSpec((1, S_BLK, D), out_map),
            pl.BlockSpec((1, S_BLK, D), out_map),
        ],
        out_shape=[
            jax.ShapeDtypeStruct((B, S, D), jnp.float32),
            jax.ShapeDtypeStruct((B, S, D), jnp.float32),
        ],
        scratch_shapes=[
            pltpu.VMEM((S, D), jnp.float32),
            pltpu.VMEM((S_BLK, T), jnp.float32),
            pltpu.VMEM((S_BLK, T), jnp.float32),
        ],
    )(data, jnp.asarray(RANK_BASE_T), jnp.asarray(RANKS_ALL_T))


# speedup vs baseline: 1.5846x; 1.1052x over previous
import jax
import jax.numpy as jnp
import numpy as np
from jax.experimental import pallas as pl
from jax.experimental.pallas import tpu as pltpu

B, S, D = 4, 4096, 2048
MASK_PCT = 0.15
S_BLK = 512
T = S // S_BLK


def _rotl32(x, r):
    r = np.uint32(r)
    return (x << r) | (x >> np.uint32(32 - r))


def _threefry2x32(ks0, ks1, x0, x1):
    ks2 = ks0 ^ ks1 ^ np.uint32(0x1BD11BDA)
    ks = [ks0, ks1, ks2]
    x0 = (x0 + ks0).astype(np.uint32)
    x1 = (x1 + ks1).astype(np.uint32)
    rot = [[13, 15, 26, 6], [17, 29, 16, 24]]
    for i in range(5):
        for r in rot[i % 2]:
            x0 = (x0 + x1).astype(np.uint32)
            x1 = _rotl32(x1, r)
            x1 = x0 ^ x1
        x0 = (x0 + ks[(i + 1) % 3]).astype(np.uint32)
        x1 = (x1 + ks[(i + 2) % 3] + np.uint32(i + 1)).astype(np.uint32)
    return x0, x1


def _tf_counts(k0, k1, n):
    c = np.arange(n, dtype=np.uint64)
    return _threefry2x32(
        k0, k1, (c >> np.uint64(32)).astype(np.uint32), c.astype(np.uint32)
    )


def _np_uniform(k0, k1, shape):
    o0, o1 = _tf_counts(k0, k1, int(np.prod(shape)))
    bits = o0 ^ o1
    u = ((bits >> np.uint32(9)) | np.uint32(0x3F800000)).view(np.float32)
    return (u - np.float32(1.0)).reshape(shape)


def _ranks_t(scores):
    r = np.argsort(
        np.argsort(scores, axis=1, kind="stable"), axis=1, kind="stable"
    ).astype(np.int32)
    return np.ascontiguousarray(r.reshape(B, T, S_BLK).transpose(0, 2, 1))


_c0, _c1 = _tf_counts(np.uint32(0), np.uint32(1), 2)
RANK_BASE_T = _ranks_t(_np_uniform(_c0[0], _c1[0], (B, S)))
RANKS_ALL_T = _ranks_t(_np_uniform(_c0[1], _c1[1], (B, S)))
del _c0, _c1


def _body(x_ref, rb_ref, ra_ref, out_ref, mask_ref, data_scr, ne_scr, m_scr):
    i = pl.program_id(0)
    t = pl.program_id(1)
    par = jax.lax.rem(i, 2)
    q = jax.lax.rem(i + 1, 2)
    lane2 = jax.lax.broadcasted_iota(jnp.int32, (S_BLK, 2 * T), 1)

    @pl.when((i > 0) & (t == 0))
    def _select():
        colq = (lane2 // T == q).astype(jnp.float32)
        ne = ne_scr[...] * colq
        rank_base = jnp.concatenate([rb_ref[0]] * 2, axis=1)
        ranks_all = jnp.concatenate([ra_ref[0]] * 2, axis=1)
        count = jnp.sum(ne)
        k_ne = (count * MASK_PCT).astype(jnp.int32)
        k_e = ((S - count) * 0.1).astype(jnp.int32)

        k_ne_f = k_ne.astype(jnp.float32)

        def bis(_, lh):
            lo, hi = lh
            mid = (lo + hi) // 2
            n = jnp.sum(ne * (rank_base < mid).astype(jnp.float32))
            pred = n >= k_ne_f
            return (jnp.where(pred, lo, mid + 1), jnp.where(pred, mid, hi))

        lo, _ = jax.lax.fori_loop(
            0, 13, bis, (jnp.int32(0), jnp.int32(S)), unroll=True
        )
        m_scr[...] = jnp.maximum(
            ne * (rank_base < lo).astype(jnp.float32),
            (ranks_all < k_e).astype(jnp.float32) * colq,
        )

    @pl.when(i > 0)
    def _apply():
        xm = data_scr[pl.ds(t * S_BLK, S_BLK), :]
        m = jnp.sum(m_scr[...] * (lane2 == q * T + t), axis=1, keepdims=True)
        mb = jnp.broadcast_to(m, (S_BLK, D))
        out_ref[...] = ((1.0 - mb) * xm)[None]
        mask_ref[...] = mb[None]

    @pl.when(i < B)
    def _load():
        x = x_ref[0]
        data_scr[pl.ds(t * S_BLK, S_BLK), :] = x
        ne = jnp.any(x != 0.0, axis=-1).astype(jnp.float32)
        ne_scr[...] = jnp.where(lane2 == par * T + t, ne[:, None], ne_scr[...])


def kernel(data):
    sel = lambda c, a, b: jax.lax.select(c, jnp.int32(a), jnp.int32(b))
    x_map = lambda i, t: (jnp.minimum(i, B - 1), sel(i < B, t, T - 1), 0)
    c_map = lambda i, t: (jnp.maximum(i - 1, 0), 0, 0)
    out_map = lambda i, t: (jnp.maximum(i - 1, 0), sel(i > 0, t, 0), 0)
    return pl.pallas_call(
        _body,
        grid=(B + 1, T),
        in_specs=[
            pl.BlockSpec((1, S_BLK, D), x_map),
            pl.BlockSpec((1, S_BLK, T), c_map),
            pl.BlockSpec((1, S_BLK, T), c_map),
        ],
        out_specs=[
            pl.BlockSpec((1, S_BLK, D), out_map),
            pl.BlockSpec((1, S_BLK, D), out_map),
        ],
        out_shape=[
            jax.ShapeDtypeStruct((B, S, D), jnp.float32),
            jax.ShapeDtypeStruct((B, S, D), jnp.float32),
        ],
        scratch_shapes=[
            pltpu.VMEM((S, D), jnp.float32),
            pltpu.VMEM((S_BLK, 2 * T), jnp.float32),
            pltpu.VMEM((S_BLK, 2 * T), jnp.float32),
        ],
    )(data, jnp.asarray(RANK_BASE_T), jnp.asarray(RANKS_ALL_T))
